# Initial kernel scaffold; baseline (speedup 1.0000x reference)
import jax
import jax.numpy as jnp
from jax.experimental import pallas as pl


def kernel(xyz, centers, masks, idx, mask_batch, W1, b1, Win, bin_, g0, be0, Wr, br, gr, betar, Wout, bout):
    raise NotImplementedError("write your pallas kernel here")


# speedup vs baseline: 1.0897x; 1.0897x over previous
import functools

import jax
import jax.numpy as jnp
from jax import lax
from jax.experimental import pallas as pl
from jax.experimental.pallas import tpu as pltpu

B, N, G, EMB, H = 2, 32768, 512, 256, 1024
BG = B * G
NTOT = B * N
P = 1024
NT = NTOT // P


def _scatter_stage(idx_sm, xyz_ref, feats_ref, idxv_ref, centers_ref,
                   w1_ref, b1_ref, out_ref, scratch_ref):
    i = pl.program_id(0)

    oh = (lax.broadcasted_iota(jnp.int32, (P, BG), 1)
          == idxv_ref[...]).astype(jnp.float32)
    cg = jnp.dot(oh, centers_ref[...], preferred_element_type=jnp.float32)
    nbh = xyz_ref[...] - cg
    dist = jnp.sqrt(jnp.sum(nbh * nbh, axis=1, keepdims=True))
    nbhn = nbh / (dist + 1e-8)

    w1 = w1_ref[...]
    feat = jnp.dot(nbhn, w1[1:4, :], preferred_element_type=jnp.float32)
    feat = feat + feats_ref[...] * w1[0:1, :]
    feat = feat + dist * w1[4:5, :]
    feat = feat + b1_ref[...]
    scratch_ref[...] = feat

    @pl.when(i == 0)
    def _():
        out_ref[...] = jnp.zeros_like(out_ref)

    base = i * P

    def body(p, carry):
        g = idx_sm[base + p]
        cur = out_ref[pl.ds(g, 1), :]
        row = scratch_ref[pl.ds(p, 1), :]
        out_ref[pl.ds(g, 1), :] = jnp.maximum(cur, row)
        return carry

    lax.fori_loop(0, P, body, 0)


def _ln(x, g, b):
    m = jnp.mean(x, axis=-1, keepdims=True)
    v = jnp.mean((x - m) * (x - m), axis=-1, keepdims=True)
    return (x - m) * lax.rsqrt(v + 1e-5) * g + b


def _gelu(x):
    return 0.5 * x * (1.0 + lax.erf(x * 0.7071067811865476))


def _mlp_stage(agg_ref, win_ref, bin_ref, g0_ref, be0_ref,
               wr_ref, br_ref, gr_ref, betar_ref, wout_ref, bout_ref, out_ref):
    x = agg_ref[...]
    h = jnp.dot(x, win_ref[...], preferred_element_type=jnp.float32)
    h = _gelu(_ln(h + bin_ref[...], g0_ref[...], be0_ref[...]))
    for l in range(wr_ref.shape[0]):
        t = jnp.dot(h, wr_ref[l], preferred_element_type=jnp.float32)
        t = _ln(t + br_ref[pl.ds(l, 1), :], gr_ref[pl.ds(l, 1), :],
                betar_ref[pl.ds(l, 1), :])
        h = h + _gelu(t)
    out_ref[...] = jnp.dot(h, wout_ref[...],
                           preferred_element_type=jnp.float32) + bout_ref[...]


@functools.partial(jax.jit, static_argnames=())
def kernel(xyz, centers, masks, idx, mask_batch, W1, b1, Win, bin_, g0, be0,
           Wr, br, gr, betar, Wout, bout):
    xyz_f = xyz.reshape(NTOT, 3)
    feats = (masks.reshape(NTOT, 1)
             * jnp.asarray(mask_batch).astype(masks.dtype))
    idx_i = idx.astype(jnp.int32)
    idx_v = idx_i.reshape(NTOT, 1)
    centers_f = centers.reshape(BG, 3)

    grid_spec = pltpu.PrefetchScalarGridSpec(
        num_scalar_prefetch=1,
        grid=(NT,),
        in_specs=[
            pl.BlockSpec((P, 3), lambda i, s: (i, 0)),
            pl.BlockSpec((P, 1), lambda i, s: (i, 0)),
            pl.BlockSpec((P, 1), lambda i, s: (i, 0)),
            pl.BlockSpec((BG, 3), lambda i, s: (0, 0)),
            pl.BlockSpec((5, H), lambda i, s: (0, 0)),
            pl.BlockSpec((1, H), lambda i, s: (0, 0)),
        ],
        out_specs=pl.BlockSpec((BG, H), lambda i, s: (0, 0)),
        scratch_shapes=[pltpu.VMEM((P, H), jnp.float32)],
    )
    agg = pl.pallas_call(
        _scatter_stage,
        grid_spec=grid_spec,
        out_shape=jax.ShapeDtypeStruct((BG, H), jnp.float32),
        compiler_params=pltpu.CompilerParams(
            dimension_semantics=("arbitrary",)),
    )(idx_i, xyz_f, feats, idx_v, centers_f, W1, b1.reshape(1, H))

    out = pl.pallas_call(
        _mlp_stage,
        out_shape=jax.ShapeDtypeStruct((BG, EMB), jnp.float32),
    )(agg, Win, bin_.reshape(1, H), g0.reshape(1, H), be0.reshape(1, H),
      Wr, br, gr, betar, Wout, bout.reshape(1, EMB))

    return out.reshape(B, G, EMB)


# speedup vs baseline: 2.5356x; 2.3269x over previous
import functools

import jax
import jax.numpy as jnp
from jax import lax
from jax.experimental import pallas as pl
from jax.experimental.pallas import tpu as pltpu

B, N, G, EMB, H = 2, 32768, 512, 256, 1024
BG = B * G
NTOT = B * N
P = 1024
NT = NTOT // P
TPB = N // P


def _scatter_stage(idx_sm, xyz_ref, feats_ref, idxv_ref, centers_ref,
                   w1_ref, out_ref, a0, a1, a2, a3, pk):
    i = pl.program_id(0)
    b = i // TPB

    loc = idxv_ref[...] - b * G
    oh = (lax.broadcasted_iota(jnp.int32, (P, G), 1)
          == loc).astype(jnp.float32)
    cg = jnp.dot(oh, centers_ref[0], preferred_element_type=jnp.float32)
    nbh = xyz_ref[...] - cg
    dist = jnp.sqrt(jnp.sum(nbh * nbh, axis=1, keepdims=True))
    nbhn = nbh / (dist + 1e-8)

    w1 = w1_ref[...]
    feat = jnp.dot(nbhn, w1[1:4, :], preferred_element_type=jnp.float32)
    feat = feat + feats_ref[...] * w1[0:1, :]
    feat = feat + dist * w1[4:5, :]
    pk[...] = feat.reshape(P, 8, 128)

    @pl.when(i == 0)
    def _():
        neg = jnp.full((BG, 8, 128), -jnp.inf, jnp.float32)
        a0[...] = neg
        a1[...] = neg
        a2[...] = neg
        a3[...] = neg

    base = i * P
    accs = (a0, a1, a2, a3)

    def body(j, carry):
        p = j * 4
        for c in range(4):
            ar = accs[c]
            g = idx_sm[base + p + c]
            ar[g] = jnp.maximum(ar[g], pk[p + c])
        return carry

    lax.fori_loop(0, P // 4, body, 0)

    @pl.when(i == NT - 1)
    def _():
        out_ref[...] = jnp.maximum(jnp.maximum(a0[...], a1[...]),
                                   jnp.maximum(a2[...], a3[...]))


def _ln(x, g, b):
    m = jnp.mean(x, axis=-1, keepdims=True)
    v = jnp.mean((x - m) * (x - m), axis=-1, keepdims=True)
    return (x - m) * lax.rsqrt(v + 1e-5) * g + b


def _gelu(x):
    return 0.5 * x * (1.0 + lax.erf(x * 0.7071067811865476))


def _mlp_stage(agg_ref, b1_ref, win_ref, bin_ref, g0_ref, be0_ref,
               wr_ref, br_ref, gr_ref, betar_ref, wout_ref, bout_ref,
               out_ref):
    x = jnp.maximum(agg_ref[...] + b1_ref[...], 0.0)
    h = jnp.dot(x, win_ref[...], preferred_element_type=jnp.float32)
    h = _gelu(_ln(h + bin_ref[...], g0_ref[...], be0_ref[...]))
    for l in range(wr_ref.shape[0]):
        t = jnp.dot(h, wr_ref[l], preferred_element_type=jnp.float32)
        t = _ln(t + br_ref[pl.ds(l, 1), :], gr_ref[pl.ds(l, 1), :],
                betar_ref[pl.ds(l, 1), :])
        h = h + _gelu(t)
    out_ref[...] = jnp.dot(h, wout_ref[...],
                           preferred_element_type=jnp.float32) + bout_ref[...]


@functools.partial(jax.jit, static_argnames=())
def kernel(xyz, centers, masks, idx, mask_batch, W1, b1, Win, bin_, g0, be0,
           Wr, br, gr, betar, Wout, bout):
    xyz_f = xyz.reshape(NTOT, 3)
    feats = (masks.reshape(NTOT, 1)
             * jnp.asarray(mask_batch).astype(masks.dtype))
    idx_i = idx.astype(jnp.int32)
    idx_v = idx_i.reshape(NTOT, 1)

    grid_spec = pltpu.PrefetchScalarGridSpec(
        num_scalar_prefetch=1,
        grid=(NT,),
        in_specs=[
            pl.BlockSpec((P, 3), lambda i, s: (i, 0)),
            pl.BlockSpec((P, 1), lambda i, s: (i, 0)),
            pl.BlockSpec((P, 1), lambda i, s: (i, 0)),
            pl.BlockSpec((1, G, 3), lambda i, s: (i // TPB, 0, 0)),
            pl.BlockSpec((5, H), lambda i, s: (0, 0)),
        ],
        out_specs=pl.BlockSpec((BG, 8, 128), lambda i, s: (0, 0, 0)),
        scratch_shapes=[pltpu.VMEM((BG, 8, 128), jnp.float32)
                        for _ in range(4)]
        + [pltpu.VMEM((P, 8, 128), jnp.float32)],
    )
    agg = pl.pallas_call(
        _scatter_stage,
        grid_spec=grid_spec,
        out_shape=jax.ShapeDtypeStruct((BG, 8, 128), jnp.float32),
        compiler_params=pltpu.CompilerParams(
            dimension_semantics=("arbitrary",)),
    )(idx_i, xyz_f, feats, idx_v, centers, W1)

    out = pl.pallas_call(
        _mlp_stage,
        out_shape=jax.ShapeDtypeStruct((BG, EMB), jnp.float32),
    )(agg.reshape(BG, H), b1.reshape(1, H), Win, bin_.reshape(1, H),
      g0.reshape(1, H), be0.reshape(1, H), Wr, br, gr, betar, Wout,
      bout.reshape(1, EMB))

    return out.reshape(B, G, EMB)
